# baseline (speedup 1.0000x reference)
import jax
import jax.numpy as jnp
import numpy as np
from jax.experimental import pallas as pl
from jax.experimental.pallas import tpu as pltpu

_IN = 4096
_OUT = 32768
_K = 1638
_TILE = 512
_GRID = _OUT // _TILE
_NP = 127


def _key_bits_to_float(m):
    i = m ^ 0x7FFFFFFF if m < 0 else m
    return np.int32(np.uint32(i & 0xFFFFFFFF).view(np.int32)).view(np.float32)


_PREFIX_F = [float(_key_bits_to_float((p - 64) << 25)) for p in range(1, 128)]


def _key_to_f32(k):
    bits = k ^ ((k >> 31) & jnp.int32(0x7FFFFFFF))
    return jax.lax.bitcast_convert_type(bits, jnp.float32)


def _fused_kernel(x_ref, w1_ref, w2_ref, b_ref, o_ref, cnt_ref):
    j = pl.program_id(0)

    @pl.when(j == 0)
    def _init():
        cnt_ref[...] = jnp.zeros(cnt_ref.shape, jnp.float32)

    dn = (((1,), (1,)), ((), ()))
    y1 = jax.lax.dot_general(x_ref[...], w1_ref[...], dimension_numbers=dn,
                             preferred_element_type=jnp.float32)
    y2 = jax.lax.dot_general(x_ref[...], w2_ref[...], dimension_numbers=dn,
                             preferred_element_type=jnp.float32)
    y = jnp.concatenate([y1, y2], axis=1) + b_ref[...]
    o_ref[:, pl.ds(j * _TILE, _TILE)] = y
    for p in range(_NP):
        tf = jnp.float32(_PREFIX_F[p])
        acc = None
        for k in range(_TILE // 128):
            mseg = jnp.where(y[:, k * 128:(k + 1) * 128] >= tf, 1.0, 0.0)
            acc = mseg if acc is None else acc + mseg
        cnt_ref[p] += acc

    @pl.when(j == _GRID - 1)
    def _select():
        rows = o_ref.shape[0]
        npref = jnp.zeros((rows, 1), jnp.float32)
        for p in range(_NP):
            c = jnp.sum(cnt_ref[p], axis=1, keepdims=True)
            npref += jnp.where(c >= _K, 1.0, 0.0)
        t = (npref.astype(jnp.int32) - 64) << 25
        for bit in range(24, -1, -1):
            cand = t | jnp.int32(1 << bit)
            cf = _key_to_f32(cand)
            acc = None
            for i in range(8):
                p8 = jnp.where(o_ref[:, i * 4096:(i + 1) * 4096] >= cf,
                               1.0, 0.0)
                while p8.shape[1] > 128:
                    h = p8.shape[1] // 2
                    p8 = p8[:, :h] + p8[:, h:]
                acc = p8 if acc is None else acc + p8
            cnt = jnp.sum(acc, axis=1, keepdims=True)
            t = jnp.where(cnt >= _K, cand, t)
        yf = o_ref[...]
        o_ref[...] = jnp.where(yf >= _key_to_f32(t), yf, 0.0)


def kernel(x, W, b):
    b2 = b.reshape(1, _OUT)
    return pl.pallas_call(
        _fused_kernel,
        grid=(_GRID,),
        in_specs=[
            pl.BlockSpec((x.shape[0], _IN), lambda j: (0, 0)),
            pl.BlockSpec((_TILE // 2, _IN), lambda j: (2 * j, 0)),
            pl.BlockSpec((_TILE // 2, _IN), lambda j: (2 * j + 1, 0)),
            pl.BlockSpec((1, _TILE), lambda j: (0, j)),
        ],
        out_specs=pl.BlockSpec((x.shape[0], _OUT), lambda j: (0, 0)),
        out_shape=jax.ShapeDtypeStruct((x.shape[0], _OUT), jnp.float32),
        scratch_shapes=[pltpu.VMEM((_NP, x.shape[0], 128), jnp.float32)],
    )(x, W, W, b2)


# speedup vs baseline: 1.0191x; 1.0191x over previous
import jax
import jax.numpy as jnp
import numpy as np
from jax.experimental import pallas as pl
from jax.experimental.pallas import tpu as pltpu

_IN = 4096
_OUT = 32768
_K = 1638
_TILE = 512
_GRID = _OUT // _TILE
_NP = 63


def _key_bits_to_float(m):
    i = m ^ 0x7FFFFFFF if m < 0 else m
    return np.int32(np.uint32(i & 0xFFFFFFFF).view(np.int32)).view(np.float32)


_PREFIX_F = [float(_key_bits_to_float((p - 32) << 26)) for p in range(1, 64)]


def _key_to_f32(k):
    bits = k ^ ((k >> 31) & jnp.int32(0x7FFFFFFF))
    return jax.lax.bitcast_convert_type(bits, jnp.float32)


def _fused_kernel(x_ref, w1_ref, w2_ref, b_ref, o_ref, cnt_ref):
    j = pl.program_id(0)

    @pl.when(j == 0)
    def _init():
        cnt_ref[...] = jnp.zeros(cnt_ref.shape, jnp.float32)

    dn = (((1,), (1,)), ((), ()))
    y1 = jax.lax.dot_general(x_ref[...], w1_ref[...], dimension_numbers=dn,
                             preferred_element_type=jnp.float32)
    y2 = jax.lax.dot_general(x_ref[...], w2_ref[...], dimension_numbers=dn,
                             preferred_element_type=jnp.float32)
    y = jnp.concatenate([y1, y2], axis=1) + b_ref[...]
    o_ref[:, pl.ds(j * _TILE, _TILE)] = y
    for p in range(_NP):
        tf = jnp.float32(_PREFIX_F[p])
        acc = None
        for k in range(_TILE // 128):
            mseg = jnp.where(y[:, k * 128:(k + 1) * 128] >= tf, 1.0, 0.0)
            acc = mseg if acc is None else acc + mseg
        cnt_ref[p] += acc

    @pl.when(j == _GRID - 1)
    def _select():
        rows = o_ref.shape[0]
        npref = jnp.zeros((rows, 1), jnp.float32)
        for p in range(_NP):
            c = jnp.sum(cnt_ref[p], axis=1, keepdims=True)
            npref += jnp.where(c >= _K, 1.0, 0.0)
        t = (npref.astype(jnp.int32) - 32) << 26
        for bit in range(25, -1, -1):
            cand = t | jnp.int32(1 << bit)
            cf = _key_to_f32(cand)
            acc = None
            for i in range(8):
                p8 = jnp.where(o_ref[:, i * 4096:(i + 1) * 4096] >= cf,
                               1.0, 0.0)
                while p8.shape[1] > 128:
                    h = p8.shape[1] // 2
                    p8 = p8[:, :h] + p8[:, h:]
                acc = p8 if acc is None else acc + p8
            cnt = jnp.sum(acc, axis=1, keepdims=True)
            t = jnp.where(cnt >= _K, cand, t)
        yf = o_ref[...]
        o_ref[...] = jnp.where(yf >= _key_to_f32(t), yf, 0.0)


def kernel(x, W, b):
    b2 = b.reshape(1, _OUT)
    return pl.pallas_call(
        _fused_kernel,
        grid=(_GRID,),
        in_specs=[
            pl.BlockSpec((x.shape[0], _IN), lambda j: (0, 0)),
            pl.BlockSpec((_TILE // 2, _IN), lambda j: (2 * j, 0)),
            pl.BlockSpec((_TILE // 2, _IN), lambda j: (2 * j + 1, 0)),
            pl.BlockSpec((1, _TILE), lambda j: (0, j)),
        ],
        out_specs=pl.BlockSpec((x.shape[0], _OUT), lambda j: (0, 0)),
        out_shape=jax.ShapeDtypeStruct((x.shape[0], _OUT), jnp.float32),
        scratch_shapes=[pltpu.VMEM((_NP, x.shape[0], 128), jnp.float32)],
    )(x, W, W, b2)
